# baseline (speedup 1.0000x reference)
import functools

import jax
import jax.numpy as jnp
from jax import lax
from jax.experimental import pallas as pl
from jax.experimental.pallas import tpu as pltpu
from jax.experimental.pallas import tpu_sc as plsc

_NC, _NS, _L = 2, 16, 16
_NW = _NC * _NS


def _proj_body(wq, wk, wv, bq, bk, bv, f, o):
    w3 = jnp.concatenate([wq[...], wk[...], wv[...]], axis=0)
    b3 = jnp.concatenate([bq[...], bk[...], bv[...]], axis=0)[:, None]
    o[...] = (
        lax.dot_general(
            w3,
            f[...],
            dimension_numbers=(((1,), (1,)), ((), ())),
            preferred_element_type=jnp.float32,
        )
        + b3
    )


@functools.lru_cache(maxsize=None)
def _make_project(N, D):
    return pl.pallas_call(
        _proj_body,
        out_shape=jax.ShapeDtypeStruct((3, N), jnp.float32),
    )


@functools.lru_cache(maxsize=None)
def _make_sc_attend(N, E, K, D):
    EPG = _L
    EPW = E // _NW
    G = EPW // EPG
    R = EPG * K
    CH2 = D // (2 * _L)
    mesh = plsc.VectorSubcoreMesh(core_axis_name="c", subcore_axis_name="s")

    @functools.partial(
        pl.kernel,
        out_type=jax.ShapeDtypeStruct((E, D), jnp.float32),
        mesh=mesh,
        compiler_params=pltpu.CompilerParams(needs_layout_passes=False),
        scratch_types=[
            pltpu.VMEM((3, N), jnp.float32),
            pltpu.VMEM((EPW * K,), jnp.int32),
            pltpu.VMEM((R, D // 2), jnp.int32),
            pltpu.VMEM((R, D // 2), jnp.int32),
            pltpu.VMEM((EPG, D), jnp.float32),
            pltpu.VMEM((EPG, D), jnp.float32),
            pltpu.VMEM((R,), jnp.float32),
            pltpu.SemaphoreType.DMA,
            pltpu.SemaphoreType.DMA,
            pltpu.SemaphoreType.DMA,
            pltpu.SemaphoreType.DMA,
        ],
    )
    def sc_attend(fb_hbm, ed_hbm, p_hbm, out_hbm,
                  qkv_v, aidx_v, rows0_v, rows1_v, out0_v, out1_v, d_v,
                  sem_r0, sem_r1, sem_o0, sem_o1):
        wid = lax.axis_index("s") * _NC + lax.axis_index("c")
        ebase = wid * EPW
        pltpu.sync_copy(ed_hbm.at[pl.ds(ebase * K, EPW * K)], aidx_v)
        pltpu.sync_copy(p_hbm, qkv_v)

        def fire(g, rows_v, sem):
            pltpu.async_copy(
                fb_hbm.at[aidx_v.at[pl.ds(g * R, R)]], rows_v, sem
            )

        def wait_rows(g, rows_v, sem):
            pltpu.make_async_copy(
                fb_hbm.at[aidx_v.at[pl.ds(g * R, R)]], rows_v, sem
            ).wait()

        def out_ref(g):
            return out_hbm.at[pl.ds(ebase + g * EPG, EPG)]

        def compute(g, rows_v, out_v, sem_o):
            ii = lax.iota(jnp.int32, _L)
            base = g * R
            vid = [
                plsc.load_gather(aidx_v, [ii * K + (base + j)])
                for j in range(K)
            ]
            row = [jnp.full((_L,), r, jnp.int32) for r in range(3)]
            q = [plsc.load_gather(qkv_v, [row[0], vid[j]]) for j in range(K)]
            k = [plsc.load_gather(qkv_v, [row[1], vid[j]]) for j in range(K)]
            v = [plsc.load_gather(qkv_v, [row[2], vid[j]]) for j in range(K)]
            for j in range(K):
                logits = [q[j] * k[m] for m in range(K)]
                ms = [m for m in range(K) if m != j]
                mx = logits[ms[0]]
                for m in ms[1:]:
                    mx = jnp.maximum(mx, logits[m])
                s = None
                num = None
                for m in ms:
                    ex = jnp.exp(logits[m] - mx)
                    s = ex if s is None else s + ex
                    w = ex * v[m]
                    num = w if num is None else num + w
                r = num / s
                d_v[pl.ds(j * _L, _L)] = 1.0 - 2.0 / (jnp.exp(r + r) + 1.0)

            @pl.when(g >= 2)
            def _():
                pltpu.make_async_copy(out_v, out_ref(g - 2), sem_o).wait()

            @plsc.parallel_loop(0, EPG)
            def _edge(i):
                db = [
                    plsc.load_gather(
                        d_v, [jnp.full((_L,), j * _L, jnp.int32) + i]
                    )
                    for j in range(K)
                ]

                @plsc.parallel_loop(0, CH2, unroll=2)
                def _chunk(c):
                    acc0 = None
                    acc1 = None
                    for j in range(K):
                        xi = rows_v[i * K + j, pl.ds(c * _L, _L)]
                        a, b = plsc.unpack(
                            plsc.bitcast(xi, jnp.bfloat16),
                            format=plsc.PackFormat.INTERLEAVED,
                            preferred_element_type=jnp.float32,
                        )
                        pa = db[j] * a
                        pb = db[j] * b
                        acc0 = pa if acc0 is None else acc0 + pa
                        acc1 = pb if acc1 is None else acc1 + pb
                    out_v[i, pl.ds(2 * c * _L, _L)] = acc0
                    out_v[i, pl.ds((2 * c + 1) * _L, _L)] = acc1

            pltpu.async_copy(out_v, out_ref(g), sem_o)

        fire(0, rows0_v, sem_r0)
        fire(1, rows1_v, sem_r1)

        @pl.loop(0, G, step=2)
        def _group(g):
            wait_rows(g, rows0_v, sem_r0)
            compute(g, rows0_v, out0_v, sem_o0)

            @pl.when(g + 2 < G)
            def _():
                fire(g + 2, rows0_v, sem_r0)

            wait_rows(g + 1, rows1_v, sem_r1)
            compute(g + 1, rows1_v, out1_v, sem_o1)

            @pl.when(g + 3 < G)
            def _():
                fire(g + 3, rows1_v, sem_r1)

        pltpu.make_async_copy(out0_v, out_ref(G - 2), sem_o0).wait()
        pltpu.make_async_copy(out1_v, out_ref(G - 1), sem_o1).wait()

    return sc_attend


def kernel(feats, edge_dict, Wq, bq, Wk, bk, Wv, bv):
    N, D = feats.shape
    E, K = edge_dict.shape
    qkv = _make_project(N, D)(Wq, Wk, Wv, bq, bk, bv, feats)
    fb = (
        feats.reshape(N, D // 32, 2, 16)
        .transpose(0, 1, 3, 2)
        .reshape(N, D // 2, 2)
        .astype(jnp.bfloat16)
    )
    fb_i32 = lax.bitcast_convert_type(fb, jnp.int32)
    return _make_sc_attend(N, E, K, D)(fb_i32, edge_dict.reshape(-1), qkv)


# speedup vs baseline: 1.0353x; 1.0353x over previous
import functools

import jax
import jax.numpy as jnp
from jax import lax
from jax.experimental import pallas as pl
from jax.experimental.pallas import tpu as pltpu
from jax.experimental.pallas import tpu_sc as plsc

_NC, _NS, _L = 2, 16, 16
_NW = _NC * _NS


def _proj_body(wq, wk, wv, bq, bk, bv, f, o):
    w3 = jnp.concatenate([wq[...], wk[...], wv[...]], axis=0)
    b3 = jnp.concatenate([bq[...], bk[...], bv[...]], axis=0)[:, None]
    o[...] = (
        lax.dot_general(
            w3,
            f[...],
            dimension_numbers=(((1,), (1,)), ((), ())),
            preferred_element_type=jnp.float32,
        )
        + b3
    )


@functools.lru_cache(maxsize=None)
def _make_project(N, D):
    return pl.pallas_call(
        _proj_body,
        out_shape=jax.ShapeDtypeStruct((3, N), jnp.float32),
    )


@functools.lru_cache(maxsize=None)
def _make_sc_attend(N, E, K, D):
    EPG = _L
    EPW = E // _NW
    G = EPW // EPG
    R = EPG * K
    CH2 = D // (2 * _L)
    mesh = plsc.VectorSubcoreMesh(core_axis_name="c", subcore_axis_name="s")

    @functools.partial(
        pl.kernel,
        out_type=jax.ShapeDtypeStruct((E, D), jnp.float32),
        mesh=mesh,
        compiler_params=pltpu.CompilerParams(needs_layout_passes=False),
        scratch_types=[
            pltpu.VMEM((3, N), jnp.float32),
            pltpu.VMEM((EPW * K,), jnp.int32),
            pltpu.VMEM((R, D // 2), jnp.int32),
            pltpu.VMEM((R, D // 2), jnp.int32),
            pltpu.VMEM((EPG, D), jnp.float32),
            pltpu.VMEM((EPG, D), jnp.float32),
            pltpu.VMEM((R,), jnp.float32),
            pltpu.SemaphoreType.DMA,
            pltpu.SemaphoreType.DMA,
            pltpu.SemaphoreType.DMA,
            pltpu.SemaphoreType.DMA,
        ],
    )
    def sc_attend(fb_hbm, ed_hbm, p_hbm, out_hbm,
                  qkv_v, aidx_v, rows0_v, rows1_v, out0_v, out1_v, d_v,
                  sem_r0, sem_r1, sem_o0, sem_o1):
        wid = lax.axis_index("s") * _NC + lax.axis_index("c")
        ebase = wid * EPW
        pltpu.sync_copy(ed_hbm.at[pl.ds(ebase * K, EPW * K)], aidx_v)
        pltpu.sync_copy(p_hbm, qkv_v)

        def fire(g, rows_v, sem):
            pltpu.async_copy(
                fb_hbm.at[aidx_v.at[pl.ds(g * R, R)]], rows_v, sem
            )

        def wait_rows(g, rows_v, sem):
            pltpu.make_async_copy(
                fb_hbm.at[aidx_v.at[pl.ds(g * R, R)]], rows_v, sem
            ).wait()

        def out_ref(g):
            return out_hbm.at[pl.ds(ebase + g * EPG, EPG)]

        def compute(g, rows_v, out_v, sem_o):
            ii = lax.iota(jnp.int32, _L)
            base = g * R
            vid = [
                plsc.load_gather(aidx_v, [ii * K + (base + j)])
                for j in range(K)
            ]
            row = [jnp.full((_L,), r, jnp.int32) for r in range(3)]
            q = [plsc.load_gather(qkv_v, [row[0], vid[j]]) for j in range(K)]
            k = [plsc.load_gather(qkv_v, [row[1], vid[j]]) for j in range(K)]
            v = [plsc.load_gather(qkv_v, [row[2], vid[j]]) for j in range(K)]
            for j in range(K):
                logits = [q[j] * k[m] for m in range(K)]
                ms = [m for m in range(K) if m != j]
                mx = logits[ms[0]]
                for m in ms[1:]:
                    mx = jnp.maximum(mx, logits[m])
                s = None
                num = None
                for m in ms:
                    ex = jnp.exp(logits[m] - mx)
                    s = ex if s is None else s + ex
                    w = ex * v[m]
                    num = w if num is None else num + w
                r = num / s
                d_v[pl.ds(j * _L, _L)] = 1.0 - 2.0 / (jnp.exp(r + r) + 1.0)

            @pl.when(g >= 2)
            def _():
                pltpu.make_async_copy(out_v, out_ref(g - 2), sem_o).wait()

            @plsc.parallel_loop(0, EPG)
            def _edge(i):
                db = [
                    plsc.load_gather(
                        d_v, [jnp.full((_L,), j * _L, jnp.int32) + i]
                    )
                    for j in range(K)
                ]
                dbb = [
                    plsc.pack(
                        db[j], db[j], format=plsc.PackFormat.INTERLEAVED
                    )
                    for j in range(K)
                ]

                @plsc.parallel_loop(0, CH2, unroll=2)
                def _chunk(c):
                    acc = None
                    for j in range(K):
                        xi = rows_v[i * K + j, pl.ds(c * _L, _L)]
                        p = dbb[j] * plsc.bitcast(xi, jnp.bfloat16)
                        acc = p if acc is None else acc + p
                    a, b = plsc.unpack(
                        acc,
                        format=plsc.PackFormat.INTERLEAVED,
                        preferred_element_type=jnp.float32,
                    )
                    out_v[i, pl.ds(2 * c * _L, _L)] = a
                    out_v[i, pl.ds((2 * c + 1) * _L, _L)] = b

            pltpu.async_copy(out_v, out_ref(g), sem_o)

        fire(0, rows0_v, sem_r0)
        fire(1, rows1_v, sem_r1)

        @pl.loop(0, G, step=2)
        def _group(g):
            wait_rows(g, rows0_v, sem_r0)
            compute(g, rows0_v, out0_v, sem_o0)

            @pl.when(g + 2 < G)
            def _():
                fire(g + 2, rows0_v, sem_r0)

            wait_rows(g + 1, rows1_v, sem_r1)
            compute(g + 1, rows1_v, out1_v, sem_o1)

            @pl.when(g + 3 < G)
            def _():
                fire(g + 3, rows1_v, sem_r1)

        pltpu.make_async_copy(out0_v, out_ref(G - 2), sem_o0).wait()
        pltpu.make_async_copy(out1_v, out_ref(G - 1), sem_o1).wait()

    return sc_attend


def kernel(feats, edge_dict, Wq, bq, Wk, bk, Wv, bv):
    N, D = feats.shape
    E, K = edge_dict.shape
    qkv = _make_project(N, D)(Wq, Wk, Wv, bq, bk, bv, feats)
    fb = (
        feats.reshape(N, D // 32, 2, 16)
        .transpose(0, 1, 3, 2)
        .reshape(N, D // 2, 2)
        .astype(jnp.bfloat16)
    )
    fb_i32 = lax.bitcast_convert_type(fb, jnp.int32)
    return _make_sc_attend(N, E, K, D)(fb_i32, edge_dict.reshape(-1), qkv)


# speedup vs baseline: 1.2116x; 1.1703x over previous
import functools

import jax
import jax.numpy as jnp
from jax import lax
from jax.experimental import pallas as pl
from jax.experimental.pallas import tpu as pltpu
from jax.experimental.pallas import tpu_sc as plsc

_NC, _NS, _L = 2, 16, 16
_NW = _NC * _NS


def _proj_body(wq, wk, wv, bq, bk, bv, f, o):
    w3 = jnp.concatenate([wq[...], wk[...], wv[...]], axis=0)
    b3 = jnp.concatenate([bq[...], bk[...], bv[...]], axis=0)[:, None]
    o[...] = (
        lax.dot_general(
            w3,
            f[...],
            dimension_numbers=(((1,), (1,)), ((), ())),
            preferred_element_type=jnp.float32,
        )
        + b3
    )


@functools.lru_cache(maxsize=None)
def _make_project(N, D):
    return pl.pallas_call(
        _proj_body,
        out_shape=jax.ShapeDtypeStruct((3, N), jnp.float32),
    )


@functools.lru_cache(maxsize=None)
def _make_sc_attend(N, E, K, D):
    EPG = _L
    EPW = E // _NW
    G = EPW // EPG
    R = EPG * K
    CH = D // _L
    mesh = plsc.VectorSubcoreMesh(core_axis_name="c", subcore_axis_name="s")

    @functools.partial(
        pl.kernel,
        out_type=jax.ShapeDtypeStruct((E, D), jnp.float32),
        mesh=mesh,
        compiler_params=pltpu.CompilerParams(needs_layout_passes=False),
        scratch_types=[
            pltpu.VMEM((3, N), jnp.float32),
            pltpu.VMEM((EPW * K,), jnp.int32),
            pltpu.VMEM((R // 2, D), jnp.float32),
            pltpu.VMEM((R // 2, D), jnp.float32),
            pltpu.VMEM((R // 2, D), jnp.float32),
            pltpu.VMEM((R // 2, D), jnp.float32),
            pltpu.VMEM((EPG, D), jnp.float32),
            pltpu.VMEM((EPG, D), jnp.float32),
            pltpu.VMEM((R,), jnp.float32),
            pltpu.SemaphoreType.DMA,
            pltpu.SemaphoreType.DMA,
            pltpu.SemaphoreType.DMA,
            pltpu.SemaphoreType.DMA,
            pltpu.SemaphoreType.DMA,
            pltpu.SemaphoreType.DMA,
        ],
    )
    def sc_attend(feats_hbm, ed_hbm, p_hbm, out_hbm,
                  qkv_v, aidx_v, rows0_v, rows1_v, rows2_v, rows3_v,
                  out0_v, out1_v, d_v,
                  sem_r0, sem_r1, sem_r2, sem_r3, sem_o0, sem_o1):
        wid = lax.axis_index("s") * _NC + lax.axis_index("c")
        ebase = wid * EPW
        pltpu.sync_copy(ed_hbm.at[pl.ds(ebase * K, EPW * K)], aidx_v)
        pltpu.sync_copy(p_hbm, qkv_v)

        H = R // 2

        def fire(h, rows_v, sem):
            pltpu.async_copy(
                feats_hbm.at[aidx_v.at[pl.ds(h * H, H)]], rows_v, sem
            )

        def wait_rows(h, rows_v, sem):
            pltpu.make_async_copy(
                feats_hbm.at[aidx_v.at[pl.ds(h * H, H)]], rows_v, sem
            ).wait()

        def out_ref(g):
            return out_hbm.at[pl.ds(ebase + g * EPG, EPG)]

        def compute(g, rows_lo, rows_hi, out_v, sem_o):
            ii = lax.iota(jnp.int32, _L)
            base = g * R
            vid = [
                plsc.load_gather(aidx_v, [ii * K + (base + j)])
                for j in range(K)
            ]
            row = [jnp.full((_L,), r, jnp.int32) for r in range(3)]
            q = [plsc.load_gather(qkv_v, [row[0], vid[j]]) for j in range(K)]
            k = [plsc.load_gather(qkv_v, [row[1], vid[j]]) for j in range(K)]
            v = [plsc.load_gather(qkv_v, [row[2], vid[j]]) for j in range(K)]
            for j in range(K):
                logits = [q[j] * k[m] for m in range(K)]
                ms = [m for m in range(K) if m != j]
                mx = logits[ms[0]]
                for m in ms[1:]:
                    mx = jnp.maximum(mx, logits[m])
                s = None
                num = None
                for m in ms:
                    ex = jnp.exp(logits[m] - mx)
                    s = ex if s is None else s + ex
                    w = ex * v[m]
                    num = w if num is None else num + w
                r = num / s
                d_v[pl.ds(j * _L, _L)] = 1.0 - 2.0 / (jnp.exp(r + r) + 1.0)

            @pl.when(g >= 2)
            def _():
                pltpu.make_async_copy(out_v, out_ref(g - 2), sem_o).wait()

            def half(rows_v, ibase):
                @plsc.parallel_loop(0, EPG // 2)
                def _edge(i2):
                    i = i2 + ibase
                    db = [
                        plsc.load_gather(
                            d_v, [jnp.full((_L,), j * _L, jnp.int32) + i]
                        )
                        for j in range(K)
                    ]

                    @plsc.parallel_loop(0, CH, unroll=4)
                    def _chunk(c):
                        acc = None
                        for j in range(K):
                            x = rows_v[i2 * K + j, pl.ds(c * _L, _L)]
                            p = db[j] * x
                            acc = p if acc is None else acc + p
                        out_v[i, pl.ds(c * _L, _L)] = acc

            half(rows_lo, 0)
            half(rows_hi, EPG // 2)

            pltpu.async_copy(out_v, out_ref(g), sem_o)

        fire(0, rows0_v, sem_r0)
        fire(1, rows1_v, sem_r1)
        fire(2, rows2_v, sem_r2)
        fire(3, rows3_v, sem_r3)

        @pl.loop(0, G, step=2)
        def _group(g):
            wait_rows(2 * g, rows0_v, sem_r0)
            wait_rows(2 * g + 1, rows1_v, sem_r1)
            compute(g, rows0_v, rows1_v, out0_v, sem_o0)

            @pl.when(2 * g + 4 < 2 * G)
            def _():
                fire(2 * g + 4, rows0_v, sem_r0)
                fire(2 * g + 5, rows1_v, sem_r1)

            wait_rows(2 * g + 2, rows2_v, sem_r2)
            wait_rows(2 * g + 3, rows3_v, sem_r3)
            compute(g + 1, rows2_v, rows3_v, out1_v, sem_o1)

            @pl.when(2 * g + 6 < 2 * G)
            def _():
                fire(2 * g + 6, rows2_v, sem_r2)
                fire(2 * g + 7, rows3_v, sem_r3)

        pltpu.make_async_copy(out0_v, out_ref(G - 2), sem_o0).wait()
        pltpu.make_async_copy(out1_v, out_ref(G - 1), sem_o1).wait()

    return sc_attend


def kernel(feats, edge_dict, Wq, bq, Wk, bk, Wv, bv):
    N, D = feats.shape
    E, K = edge_dict.shape
    qkv = _make_project(N, D)(Wq, Wk, Wv, bq, bk, bv, feats)
    return _make_sc_attend(N, E, K, D)(feats, edge_dict.reshape(-1), qkv)


# speedup vs baseline: 1.2415x; 1.0247x over previous
import functools

import jax
import jax.numpy as jnp
from jax import lax
from jax.experimental import pallas as pl
from jax.experimental.pallas import tpu as pltpu
from jax.experimental.pallas import tpu_sc as plsc

_NC, _NS, _L = 2, 16, 16
_NW = _NC * _NS


def _proj_body(wq, wk, wv, bq, bk, bv, f, o):
    w3 = jnp.concatenate([wq[...], wk[...], wv[...]], axis=0)
    b3 = jnp.concatenate([bq[...], bk[...], bv[...]], axis=0)[:, None]
    o[...] = (
        lax.dot_general(
            w3,
            f[...],
            dimension_numbers=(((1,), (1,)), ((), ())),
            preferred_element_type=jnp.float32,
        )
        + b3
    )


@functools.lru_cache(maxsize=None)
def _make_project(N, D):
    return pl.pallas_call(
        _proj_body,
        out_shape=jax.ShapeDtypeStruct((3, N), jnp.float32),
    )


@functools.lru_cache(maxsize=None)
def _make_sc_attend(N, E, K, D):
    EPG = _L
    EPW = E // _NW
    G = EPW // EPG
    R = EPG * K
    CH = D // _L
    mesh = plsc.VectorSubcoreMesh(core_axis_name="c", subcore_axis_name="s")

    @functools.partial(
        pl.kernel,
        out_type=jax.ShapeDtypeStruct((E, D), jnp.float32),
        mesh=mesh,
        compiler_params=pltpu.CompilerParams(needs_layout_passes=False),
        scratch_types=[
            pltpu.VMEM((3, N), jnp.float32),
            pltpu.VMEM((EPW * K,), jnp.int32),
            pltpu.VMEM((R, D), jnp.float32),
            pltpu.VMEM((R, D), jnp.float32),
            pltpu.VMEM((EPG, D), jnp.float32),
            pltpu.VMEM((EPG, D), jnp.float32),
            pltpu.VMEM((R,), jnp.float32),
            pltpu.SemaphoreType.DMA,
            pltpu.SemaphoreType.DMA,
            pltpu.SemaphoreType.DMA,
            pltpu.SemaphoreType.DMA,
        ],
    )
    def sc_attend(feats_hbm, ed_hbm, p_hbm, out_hbm,
                  qkv_v, aidx_v, rows0_v, rows1_v, out0_v, out1_v, d_v,
                  sem_r0, sem_r1, sem_o0, sem_o1):
        wid = lax.axis_index("s") * _NC + lax.axis_index("c")
        ebase = wid * EPW
        pltpu.sync_copy(ed_hbm.at[pl.ds(ebase * K, EPW * K)], aidx_v)

        def fire(g, rows_v, sem):
            pltpu.async_copy(
                feats_hbm.at[aidx_v.at[pl.ds(g * R, R)]], rows_v, sem
            )

        def wait_rows(g, rows_v, sem):
            pltpu.make_async_copy(
                feats_hbm.at[aidx_v.at[pl.ds(g * R, R)]], rows_v, sem
            ).wait()

        def out_ref(g):
            return out_hbm.at[pl.ds(ebase + g * EPG, EPG)]

        def compute(g, rows_v, out_v, sem_o):
            ii = lax.iota(jnp.int32, _L)
            base = g * R
            vid = [
                plsc.load_gather(aidx_v, [ii * K + (base + j)])
                for j in range(K)
            ]
            row = [jnp.full((_L,), r, jnp.int32) for r in range(3)]
            q = [plsc.load_gather(qkv_v, [row[0], vid[j]]) for j in range(K)]
            k = [plsc.load_gather(qkv_v, [row[1], vid[j]]) for j in range(K)]
            v = [plsc.load_gather(qkv_v, [row[2], vid[j]]) for j in range(K)]
            for j in range(K):
                logits = [q[j] * k[m] for m in range(K)]
                ms = [m for m in range(K) if m != j]
                mx = logits[ms[0]]
                for m in ms[1:]:
                    mx = jnp.maximum(mx, logits[m])
                s = None
                num = None
                for m in ms:
                    ex = jnp.exp(logits[m] - mx)
                    s = ex if s is None else s + ex
                    w = ex * v[m]
                    num = w if num is None else num + w
                r = num / s
                d_v[pl.ds(j * _L, _L)] = 1.0 - 2.0 / (jnp.exp(r + r) + 1.0)

            @pl.when(g >= 2)
            def _():
                pltpu.make_async_copy(out_v, out_ref(g - 2), sem_o).wait()

            @plsc.parallel_loop(0, EPG)
            def _edge(i):
                db = [
                    plsc.load_gather(
                        d_v, [jnp.full((_L,), j * _L, jnp.int32) + i]
                    )
                    for j in range(K)
                ]

                @plsc.parallel_loop(0, CH, unroll=4)
                def _chunk(c):
                    acc = db[0] * rows_v[i * K, pl.ds(c * _L, _L)]
                    for j in range(1, K):
                        acc = acc + db[j] * rows_v[i * K + j, pl.ds(c * _L, _L)]
                    out_v[i, pl.ds(c * _L, _L)] = acc

            pltpu.async_copy(out_v, out_ref(g), sem_o)

        fire(0, rows0_v, sem_r0)
        fire(1, rows1_v, sem_r1)
        pltpu.sync_copy(p_hbm, qkv_v)

        @pl.loop(0, G, step=2)
        def _group(g):
            wait_rows(g, rows0_v, sem_r0)
            compute(g, rows0_v, out0_v, sem_o0)

            @pl.when(g + 2 < G)
            def _():
                fire(g + 2, rows0_v, sem_r0)

            wait_rows(g + 1, rows1_v, sem_r1)
            compute(g + 1, rows1_v, out1_v, sem_o1)

            @pl.when(g + 3 < G)
            def _():
                fire(g + 3, rows1_v, sem_r1)

        pltpu.make_async_copy(out0_v, out_ref(G - 2), sem_o0).wait()
        pltpu.make_async_copy(out1_v, out_ref(G - 1), sem_o1).wait()

    return sc_attend


def kernel(feats, edge_dict, Wq, bq, Wk, bk, Wv, bv):
    N, D = feats.shape
    E, K = edge_dict.shape
    qkv = _make_project(N, D)(Wq, Wk, Wv, bq, bk, bv, feats)
    return _make_sc_attend(N, E, K, D)(feats, edge_dict.reshape(-1), qkv)


# speedup vs baseline: 1.2566x; 1.0122x over previous
import functools

import jax
import jax.numpy as jnp
from jax import lax
from jax.experimental import pallas as pl
from jax.experimental.pallas import tpu as pltpu
from jax.experimental.pallas import tpu_sc as plsc

_NC, _NS, _L = 2, 16, 16
_NW = _NC * _NS


def _proj_body(wq, wk, wv, bq, bk, bv, f, o):
    w3 = jnp.concatenate([wq[...], wk[...], wv[...]], axis=0)
    b3 = jnp.concatenate([bq[...], bk[...], bv[...]], axis=0)[:, None]
    o[...] = (
        lax.dot_general(
            w3,
            f[...],
            dimension_numbers=(((1,), (1,)), ((), ())),
            preferred_element_type=jnp.float32,
        )
        + b3
    )


@functools.lru_cache(maxsize=None)
def _make_project(N, D):
    return pl.pallas_call(
        _proj_body,
        out_shape=jax.ShapeDtypeStruct((3, N), jnp.float32),
    )


@functools.lru_cache(maxsize=None)
def _make_sc_attend(N, E, K, D):
    EPG = _L
    EPW = E // _NW
    G = EPW // EPG
    R = EPG * K
    CH = D // _L
    mesh = plsc.VectorSubcoreMesh(core_axis_name="c", subcore_axis_name="s")

    @functools.partial(
        pl.kernel,
        out_type=jax.ShapeDtypeStruct((E, D), jnp.float32),
        mesh=mesh,
        compiler_params=pltpu.CompilerParams(needs_layout_passes=False),
        scratch_types=[
            pltpu.VMEM((3, N), jnp.float32),
            pltpu.VMEM((EPW * K,), jnp.int32),
            pltpu.VMEM((R, D), jnp.float32),
            pltpu.VMEM((R, D), jnp.float32),
            pltpu.VMEM((EPG, D), jnp.float32),
            pltpu.VMEM((EPG, D), jnp.float32),
            pltpu.VMEM((R,), jnp.float32),
            pltpu.SemaphoreType.DMA,
            pltpu.SemaphoreType.DMA,
            pltpu.SemaphoreType.DMA,
            pltpu.SemaphoreType.DMA,
        ],
    )
    def sc_attend(feats_hbm, ed_hbm, p_hbm, out_hbm,
                  qkv_v, aidx_v, rows0_v, rows1_v, out0_v, out1_v, d_v,
                  sem_r0, sem_r1, sem_o0, sem_o1):
        wid = lax.axis_index("s") * _NC + lax.axis_index("c")
        ebase = wid * EPW
        pltpu.sync_copy(ed_hbm.at[pl.ds(ebase * K, EPW * K)], aidx_v)

        def fire(g, rows_v, sem):
            pltpu.async_copy(
                feats_hbm.at[aidx_v.at[pl.ds(g * R, R)]], rows_v, sem
            )

        def wait_rows(g, rows_v, sem):
            pltpu.make_async_copy(
                feats_hbm.at[aidx_v.at[pl.ds(g * R, R)]], rows_v, sem
            ).wait()

        def out_ref(g):
            return out_hbm.at[pl.ds(ebase + g * EPG, EPG)]

        def compute(g, rows_v, out_v, sem_o, sem_r):
            ii = lax.iota(jnp.int32, _L)
            base = g * R
            vid = [
                plsc.load_gather(aidx_v, [ii * K + (base + j)])
                for j in range(K)
            ]
            row = [jnp.full((_L,), r, jnp.int32) for r in range(3)]
            q = [plsc.load_gather(qkv_v, [row[0], vid[j]]) for j in range(K)]
            k = [plsc.load_gather(qkv_v, [row[1], vid[j]]) for j in range(K)]
            v = [plsc.load_gather(qkv_v, [row[2], vid[j]]) for j in range(K)]
            for j in range(K):
                logits = [q[j] * k[m] for m in range(K)]
                ms = [m for m in range(K) if m != j]
                mx = logits[ms[0]]
                for m in ms[1:]:
                    mx = jnp.maximum(mx, logits[m])
                s = None
                num = None
                for m in ms:
                    ex = jnp.exp(logits[m] - mx)
                    s = ex if s is None else s + ex
                    w = ex * v[m]
                    num = w if num is None else num + w
                r = num / s
                d_v[pl.ds(j * _L, _L)] = 1.0 - 2.0 / (jnp.exp(r + r) + 1.0)

            wait_rows(g, rows_v, sem_r)

            @pl.when(g >= 2)
            def _():
                pltpu.make_async_copy(out_v, out_ref(g - 2), sem_o).wait()

            @plsc.parallel_loop(0, EPG)
            def _edge(i):
                db = [
                    plsc.load_gather(
                        d_v, [jnp.full((_L,), j * _L, jnp.int32) + i]
                    )
                    for j in range(K)
                ]

                @plsc.parallel_loop(0, CH, unroll=4)
                def _chunk(c):
                    acc = db[0] * rows_v[i * K, pl.ds(c * _L, _L)]
                    for j in range(1, K):
                        acc = acc + db[j] * rows_v[i * K + j, pl.ds(c * _L, _L)]
                    out_v[i, pl.ds(c * _L, _L)] = acc

            pltpu.async_copy(out_v, out_ref(g), sem_o)

        fire(0, rows0_v, sem_r0)
        fire(1, rows1_v, sem_r1)
        pltpu.sync_copy(p_hbm, qkv_v)

        @pl.loop(0, G, step=2)
        def _group(g):
            compute(g, rows0_v, out0_v, sem_o0, sem_r0)

            @pl.when(g + 2 < G)
            def _():
                fire(g + 2, rows0_v, sem_r0)

            compute(g + 1, rows1_v, out1_v, sem_o1, sem_r1)

            @pl.when(g + 3 < G)
            def _():
                fire(g + 3, rows1_v, sem_r1)

        pltpu.make_async_copy(out0_v, out_ref(G - 2), sem_o0).wait()
        pltpu.make_async_copy(out1_v, out_ref(G - 1), sem_o1).wait()

    return sc_attend


def kernel(feats, edge_dict, Wq, bq, Wk, bk, Wv, bv):
    N, D = feats.shape
    E, K = edge_dict.shape
    qkv = _make_project(N, D)(Wq, Wk, Wv, bq, bk, bv, feats)
    return _make_sc_attend(N, E, K, D)(feats, edge_dict.reshape(-1), qkv)


# speedup vs baseline: 1.2736x; 1.0135x over previous
import functools

import jax
import jax.numpy as jnp
from jax import lax
from jax.experimental import pallas as pl
from jax.experimental.pallas import tpu as pltpu
from jax.experimental.pallas import tpu_sc as plsc

_NC, _NS, _L = 2, 16, 16
_NW = _NC * _NS


def _proj_body(wq, wk, wv, bq, bk, bv, f, o):
    i = pl.program_id(0)
    w3 = jnp.concatenate([wq[...], wk[...], wv[...]], axis=0)
    b3 = jnp.concatenate([bq[...], bk[...], bv[...]], axis=0)[:, None]
    part = lax.dot_general(
        w3,
        f[...],
        dimension_numbers=(((1,), (1,)), ((), ())),
        preferred_element_type=jnp.float32,
    )

    @pl.when(i == 0)
    def _():
        o[...] = part + b3

    @pl.when(i != 0)
    def _():
        o[...] = o[...] + part


@functools.lru_cache(maxsize=None)
def _make_project(N, D):
    BD = 128
    return pl.pallas_call(
        _proj_body,
        grid=(D // BD,),
        in_specs=[
            pl.BlockSpec((1, BD), lambda i: (0, i)),
            pl.BlockSpec((1, BD), lambda i: (0, i)),
            pl.BlockSpec((1, BD), lambda i: (0, i)),
            pl.BlockSpec((1,), lambda i: (0,)),
            pl.BlockSpec((1,), lambda i: (0,)),
            pl.BlockSpec((1,), lambda i: (0,)),
            pl.BlockSpec((N, BD), lambda i: (0, i)),
        ],
        out_specs=pl.BlockSpec((3, N), lambda i: (0, 0)),
        out_shape=jax.ShapeDtypeStruct((3, N), jnp.float32),
    )


@functools.lru_cache(maxsize=None)
def _make_sc_attend(N, E, K, D):
    EPG = _L
    EPW = E // _NW
    G = EPW // EPG
    R = EPG * K
    CH = D // _L
    mesh = plsc.VectorSubcoreMesh(core_axis_name="c", subcore_axis_name="s")

    @functools.partial(
        pl.kernel,
        out_type=jax.ShapeDtypeStruct((E, D), jnp.float32),
        mesh=mesh,
        compiler_params=pltpu.CompilerParams(needs_layout_passes=False),
        scratch_types=[
            pltpu.VMEM((3, N), jnp.float32),
            pltpu.VMEM((EPW * K,), jnp.int32),
            pltpu.VMEM((R, D), jnp.float32),
            pltpu.VMEM((R, D), jnp.float32),
            pltpu.VMEM((EPG, D), jnp.float32),
            pltpu.VMEM((EPG, D), jnp.float32),
            pltpu.VMEM((R,), jnp.float32),
            pltpu.SemaphoreType.DMA,
            pltpu.SemaphoreType.DMA,
            pltpu.SemaphoreType.DMA,
            pltpu.SemaphoreType.DMA,
        ],
    )
    def sc_attend(feats_hbm, ed_hbm, p_hbm, out_hbm,
                  qkv_v, aidx_v, rows0_v, rows1_v, out0_v, out1_v, d_v,
                  sem_r0, sem_r1, sem_o0, sem_o1):
        wid = lax.axis_index("s") * _NC + lax.axis_index("c")
        ebase = wid * EPW
        pltpu.sync_copy(ed_hbm.at[pl.ds(ebase * K, EPW * K)], aidx_v)

        def fire(g, rows_v, sem):
            pltpu.async_copy(
                feats_hbm.at[aidx_v.at[pl.ds(g * R, R)]], rows_v, sem
            )

        def wait_rows(g, rows_v, sem):
            pltpu.make_async_copy(
                feats_hbm.at[aidx_v.at[pl.ds(g * R, R)]], rows_v, sem
            ).wait()

        def out_ref(g):
            return out_hbm.at[pl.ds(ebase + g * EPG, EPG)]

        def compute(g, rows_v, out_v, sem_o, sem_r):
            ii = lax.iota(jnp.int32, _L)
            base = g * R
            vid = [
                plsc.load_gather(aidx_v, [ii * K + (base + j)])
                for j in range(K)
            ]
            row = [jnp.full((_L,), r, jnp.int32) for r in range(3)]
            q = [plsc.load_gather(qkv_v, [row[0], vid[j]]) for j in range(K)]
            k = [plsc.load_gather(qkv_v, [row[1], vid[j]]) for j in range(K)]
            v = [plsc.load_gather(qkv_v, [row[2], vid[j]]) for j in range(K)]
            for j in range(K):
                logits = [q[j] * k[m] for m in range(K)]
                ms = [m for m in range(K) if m != j]
                mx = logits[ms[0]]
                for m in ms[1:]:
                    mx = jnp.maximum(mx, logits[m])
                s = None
                num = None
                for m in ms:
                    ex = jnp.exp(logits[m] - mx)
                    s = ex if s is None else s + ex
                    w = ex * v[m]
                    num = w if num is None else num + w
                r = num / s
                d_v[pl.ds(j * _L, _L)] = 1.0 - 2.0 / (jnp.exp(r + r) + 1.0)

            wait_rows(g, rows_v, sem_r)

            @pl.when(g >= 2)
            def _():
                pltpu.make_async_copy(out_v, out_ref(g - 2), sem_o).wait()

            @plsc.parallel_loop(0, EPG)
            def _edge(i):
                db = [
                    plsc.load_gather(
                        d_v, [jnp.full((_L,), j * _L, jnp.int32) + i]
                    )
                    for j in range(K)
                ]

                @plsc.parallel_loop(0, CH, unroll=4)
                def _chunk(c):
                    acc = db[0] * rows_v[i * K, pl.ds(c * _L, _L)]
                    for j in range(1, K):
                        acc = acc + db[j] * rows_v[i * K + j, pl.ds(c * _L, _L)]
                    out_v[i, pl.ds(c * _L, _L)] = acc

            pltpu.async_copy(out_v, out_ref(g), sem_o)

        fire(0, rows0_v, sem_r0)
        fire(1, rows1_v, sem_r1)
        pltpu.sync_copy(p_hbm, qkv_v)

        @pl.loop(0, G, step=2)
        def _group(g):
            compute(g, rows0_v, out0_v, sem_o0, sem_r0)

            @pl.when(g + 2 < G)
            def _():
                fire(g + 2, rows0_v, sem_r0)

            compute(g + 1, rows1_v, out1_v, sem_o1, sem_r1)

            @pl.when(g + 3 < G)
            def _():
                fire(g + 3, rows1_v, sem_r1)

        pltpu.make_async_copy(out0_v, out_ref(G - 2), sem_o0).wait()
        pltpu.make_async_copy(out1_v, out_ref(G - 1), sem_o1).wait()

    return sc_attend


def kernel(feats, edge_dict, Wq, bq, Wk, bk, Wv, bv):
    N, D = feats.shape
    E, K = edge_dict.shape
    qkv = _make_project(N, D)(Wq, Wk, Wv, bq, bk, bv, feats)
    return _make_sc_attend(N, E, K, D)(feats, edge_dict.reshape(-1), qkv)
